# baseline (speedup 1.0000x reference)
import jax
import jax.numpy as jnp
from jax.experimental import pallas as pl

_BLK_I = 400


def _support_body(x_ref, w1_ref, out_ref):
    out_ref[...] = jnp.dot(
        x_ref[...], w1_ref[...],
        precision=jax.lax.Precision.HIGHEST,
        preferred_element_type=jnp.float32,
    ).astype(jnp.bfloat16)


def _mid_body(adj_ref, sup_ref, w2_ref, out_ref):
    acc = jnp.dot(
        adj_ref[...].astype(jnp.bfloat16), sup_ref[...],
        preferred_element_type=jnp.float32,
    )
    h = jnp.maximum(acc, 0.0)
    out_ref[...] = jnp.dot(
        h, w2_ref[...],
        precision=jax.lax.Precision.HIGHEST,
        preferred_element_type=jnp.float32,
    ).astype(jnp.bfloat16)


def _out_body(adj_ref, s2_ref, out_ref):
    out_ref[...] = jnp.dot(
        adj_ref[...].astype(jnp.bfloat16), s2_ref[...],
        preferred_element_type=jnp.float32,
    )


def kernel(x, adjacency, W1, W2):
    N, D = x.shape
    blk = _BLK_I
    grid = (N // blk,)

    support = pl.pallas_call(
        _support_body,
        grid=(N // 2000,),
        in_specs=[
            pl.BlockSpec((2000, D), lambda i: (i, 0)),
            pl.BlockSpec((D, D), lambda i: (0, 0)),
        ],
        out_specs=pl.BlockSpec((2000, D), lambda i: (i, 0)),
        out_shape=jax.ShapeDtypeStruct((N, D), jnp.bfloat16),
    )(x, W1)

    s2 = pl.pallas_call(
        _mid_body,
        grid=grid,
        in_specs=[
            pl.BlockSpec((blk, N), lambda i: (i, 0)),
            pl.BlockSpec((N, D), lambda i: (0, 0)),
            pl.BlockSpec((D, D), lambda i: (0, 0)),
        ],
        out_specs=pl.BlockSpec((blk, D), lambda i: (i, 0)),
        out_shape=jax.ShapeDtypeStruct((N, D), jnp.bfloat16),
    )(adjacency, support, W2)

    logits = pl.pallas_call(
        _out_body,
        grid=grid,
        in_specs=[
            pl.BlockSpec((blk, N), lambda i: (i, 0)),
            pl.BlockSpec((N, D), lambda i: (0, 0)),
        ],
        out_specs=pl.BlockSpec((blk, D), lambda i: (i, 0)),
        out_shape=jax.ShapeDtypeStruct((N, D), jnp.float32),
    )(adjacency, s2)

    return logits
